# baseline (speedup 1.0000x reference)
import functools

import jax
import jax.numpy as jnp
from jax import lax
from jax.experimental import pallas as pl
from jax.experimental.pallas import tpu as pltpu
from jax.experimental.pallas import tpu_sc as plsc


@functools.lru_cache(maxsize=None)
def _make_gather(V, D, B):
    info = plsc.get_sparse_core_info()
    NC, NS = info.num_cores, info.num_subcores
    NW = NC * NS
    assert B % NW == 0
    b_per_w = B // NW
    CH = 512
    assert b_per_w % CH == 0
    NCH = b_per_w // CH

    mesh = plsc.VectorSubcoreMesh(core_axis_name="c", subcore_axis_name="s")

    @functools.partial(
        pl.kernel,
        mesh=mesh,
        compiler_params=pltpu.CompilerParams(use_tc_tiling_on_sc=False),
        out_type=jax.ShapeDtypeStruct((B, D), jnp.float32),
        scratch_types=[
            pltpu.VMEM((CH,), jnp.int32),
            pltpu.VMEM((CH, D), jnp.float32),
            pltpu.SemaphoreType.DMA,
        ],
    )
    def k(idx_hbm, table_hbm, out_hbm, idx_v, rows_v, sem):
        wid = lax.axis_index("s") * NC + lax.axis_index("c")
        base = wid * b_per_w

        def body(g, carry):
            off = base + g * CH
            pltpu.sync_copy(idx_hbm.at[pl.ds(off, CH)], idx_v)
            pltpu.async_copy(table_hbm.at[idx_v], rows_v, sem).wait()
            pltpu.sync_copy(rows_v, out_hbm.at[pl.ds(off, CH)])
            return carry

        lax.fori_loop(0, NCH, body, 0)

    return k


def kernel(x, table):
    BATCH, HIST = x.shape
    V, D = table.shape
    flat = x.reshape(BATCH * HIST)
    out = _make_gather(V, D, BATCH * HIST)(flat, table)
    return out.reshape(BATCH, HIST, D)


# speedup vs baseline: 1.0477x; 1.0477x over previous
import functools

import jax
import jax.numpy as jnp
from jax import lax
from jax.experimental import pallas as pl
from jax.experimental.pallas import tpu as pltpu
from jax.experimental.pallas import tpu_sc as plsc

_CH = 320
_NBUF = 4
_LA = 2


@functools.lru_cache(maxsize=None)
def _make_gather(V, D, B):
    info = plsc.get_sparse_core_info()
    NC, NS = info.num_cores, info.num_subcores
    NW = NC * NS
    assert B % NW == 0
    b_per_w = B // NW
    CH, NBUF, LA = _CH, _NBUF, _LA
    assert b_per_w % (CH * NBUF) == 0
    NCH = b_per_w // CH

    mesh = plsc.VectorSubcoreMesh(core_axis_name="c", subcore_axis_name="s")

    @functools.partial(
        pl.kernel,
        mesh=mesh,
        compiler_params=pltpu.CompilerParams(use_tc_tiling_on_sc=False),
        out_type=jax.ShapeDtypeStruct((B, D), jnp.float32),
        scratch_types=[
            pltpu.VMEM((b_per_w,), jnp.int32),
            [pltpu.VMEM((CH, D), jnp.float32) for _ in range(NBUF)],
            [pltpu.SemaphoreType.DMA for _ in range(NBUF)],
            [pltpu.SemaphoreType.DMA for _ in range(NBUF)],
        ],
    )
    def k(idx_hbm, table_hbm, out_hbm, idx_v, rows, gsem, osem):
        wid = lax.axis_index("s") * NC + lax.axis_index("c")
        base = wid * b_per_w

        pltpu.sync_copy(idx_hbm.at[pl.ds(base, b_per_w)], idx_v)

        def gather(chunk, slot):
            pltpu.async_copy(
                table_hbm.at[idx_v.at[pl.ds(chunk * CH, CH)]],
                rows[slot], gsem[slot])

        def writeout(chunk, slot):
            pltpu.async_copy(
                rows[slot], out_hbm.at[pl.ds(base + chunk * CH, CH)],
                osem[slot])

        for c in range(LA):
            gather(c, c)

        def group(gg, carry):
            for b in range(NBUF):
                g = gg * NBUF + b
                gn = g + LA
                sn = (b + LA) % NBUF

                @pl.when(gn < NCH)
                def _():
                    @pl.when(gn >= NBUF)
                    def _():
                        pltpu.make_async_copy(
                            rows[sn],
                            out_hbm.at[pl.ds(base + (gn - NBUF) * CH, CH)],
                            osem[sn]).wait()
                    gather(gn, sn)

                pltpu.make_async_copy(
                    table_hbm.at[idx_v.at[pl.ds(g * CH, CH)]],
                    rows[b], gsem[b]).wait()
                writeout(g, b)
            return carry

        lax.fori_loop(0, NCH // NBUF, group, 0)

        for b in range(NBUF):
            g = NCH - NBUF + b
            pltpu.make_async_copy(
                rows[b], out_hbm.at[pl.ds(base + g * CH, CH)],
                osem[b]).wait()

    return k


def kernel(x, table):
    BATCH, HIST = x.shape
    V, D = table.shape
    flat = x.reshape(BATCH * HIST)
    out = _make_gather(V, D, BATCH * HIST)(flat, table)
    return out.reshape(BATCH, HIST, D)
